# Initial kernel scaffold; baseline (speedup 1.0000x reference)
import jax
import jax.numpy as jnp
from jax.experimental import pallas as pl


def kernel(x_z, x_atom, edge_index, edge_attr, batch, emb_z, atom_in_W, atom_in_b, bond_in_W, bond_in_b, W_msg, gru_W_ih, gru_W_hh, gru_b_ih, gru_b_hh, atom_out_W, atom_out_b, readout_W, readout_b, ln_e_w, ln_e_b, ln_a_w, ln_a_b):
    raise NotImplementedError("write your pallas kernel here")


# speedup vs baseline: 31.5245x; 31.5245x over previous
import functools

import jax
import jax.numpy as jnp
from jax import lax
from jax.experimental import pallas as pl
from jax.experimental.pallas import tpu as pltpu

N = 10000
E = 320000
E2 = 2 * E
H = 128
G = 256
DEPTH = 3
TASKS = 12

EDGE_BLOCK = 2560


def _ln(x, w, b):
    mu = jnp.mean(x, axis=-1, keepdims=True)
    var = jnp.mean((x - mu) ** 2, axis=-1, keepdims=True)
    return (x - mu) / jnp.sqrt(var + 1e-5) * w + b


def _depth_step_body(h_ref, mx_ref, wmsg_ref, wih_ref, whh_ref, bih_ref, bhh_ref,
                     lnw_ref, lnb_ref, out_ref):
    h_e = h_ref[...]
    m = jnp.dot(mx_ref[...], wmsg_ref[...], preferred_element_type=jnp.float32)
    h_new = jnp.maximum(h_e + m, 0.0)
    gi = jnp.dot(h_new, wih_ref[...], preferred_element_type=jnp.float32) + bih_ref[...]
    gh = jnp.dot(h_e, whh_ref[...], preferred_element_type=jnp.float32) + bhh_ref[...]
    r = jax.nn.sigmoid(gi[:, :H] + gh[:, :H])
    z = jax.nn.sigmoid(gi[:, H:2 * H] + gh[:, H:2 * H])
    ng = jnp.tanh(gi[:, 2 * H:] + r * gh[:, 2 * H:])
    h = (1.0 - z) * ng + z * h_e
    mu = jnp.mean(h, axis=-1, keepdims=True)
    var = jnp.mean((h - mu) ** 2, axis=-1, keepdims=True)
    out_ref[...] = (h - mu) * jax.lax.rsqrt(var + 1e-5) * lnw_ref[...] + lnb_ref[...]


def _depth_step(h_e, m_excl, wmsg_t, wih_t, whh_t, bih, bhh, lnw, lnb):
    nblk = E2 // EDGE_BLOCK
    blk = lambda i: (i, i * 0)
    full = lambda i: (i * 0, i * 0)
    call = pl.pallas_call(
        _depth_step_body,
        grid=(nblk,),
        in_specs=[
            pl.BlockSpec((EDGE_BLOCK, H), blk),
            pl.BlockSpec((EDGE_BLOCK, H), blk),
            pl.BlockSpec((H, H), full),
            pl.BlockSpec((H, 3 * H), full),
            pl.BlockSpec((H, 3 * H), full),
            pl.BlockSpec((1, 3 * H), full),
            pl.BlockSpec((1, 3 * H), full),
            pl.BlockSpec((1, H), full),
            pl.BlockSpec((1, H), full),
        ],
        out_specs=pl.BlockSpec((EDGE_BLOCK, H), blk),
        out_shape=jax.ShapeDtypeStruct((E2, H), jnp.float32),
    )
    return call(h_e, m_excl, wmsg_t, wih_t, whh_t, bih, bhh, lnw, lnb)


def _build_directed(edge_index):
    src_u, dst_u = edge_index[0], edge_index[1]
    src_dir = jnp.concatenate([src_u, dst_u], axis=0)
    dst_dir = jnp.concatenate([dst_u, src_u], axis=0)
    key = (src_dir.astype(jnp.int64) << 32) + dst_dir.astype(jnp.int64)
    rev_key = (dst_dir.astype(jnp.int64) << 32) + src_dir.astype(jnp.int64)
    order = jnp.argsort(key)
    key_sorted = key[order]
    idx_in_sorted = jnp.searchsorted(key_sorted, rev_key)
    rev = order[idx_in_sorted]
    return src_dir.astype(jnp.int32), dst_dir.astype(jnp.int32), rev.astype(jnp.int32)


def kernel(x_z, x_atom, edge_index, edge_attr, batch, emb_z, atom_in_W, atom_in_b,
           bond_in_W, bond_in_b, W_msg, gru_W_ih, gru_W_hh, gru_b_ih, gru_b_hh,
           atom_out_W, atom_out_b, readout_W, readout_b, ln_e_w, ln_e_b, ln_a_w, ln_a_b):
    src_dir, dst_dir, rev = _build_directed(edge_index)

    atom_in_W = atom_in_W.astype(jnp.float32)
    bond_in_W = bond_in_W.astype(jnp.float32)
    W_msg = W_msg.astype(jnp.float32)
    gru_W_ih = gru_W_ih.astype(jnp.float32)
    gru_W_hh = gru_W_hh.astype(jnp.float32)
    atom_out_W = atom_out_W.astype(jnp.float32)
    readout_W = readout_W.astype(jnp.float32)

    a_embed = jnp.take(emb_z, x_z, axis=0)
    a_all = jnp.concatenate([a_embed, x_atom], axis=1)
    h_all = a_all @ atom_in_W.T + atom_in_b
    e_all = edge_attr @ bond_in_W.T + bond_in_b
    h_src = jnp.take(h_all, src_dir, axis=0)
    e_in = jnp.concatenate([e_all, e_all], axis=0)
    h_e = jax.nn.relu(h_src + e_in)
    h_e = _ln(h_e, ln_e_w, ln_e_b)

    wmsg_t = W_msg.T
    wih_t = gru_W_ih.T
    whh_t = gru_W_hh.T
    bih = gru_b_ih.reshape(1, 3 * H)
    bhh = gru_b_hh.reshape(1, 3 * H)
    lnw = ln_e_w.reshape(1, H)
    lnb = ln_e_b.reshape(1, H)

    for _ in range(DEPTH):
        m_in = jax.ops.segment_sum(h_e, dst_dir, num_segments=N)
        m_excl = jnp.take(m_in, src_dir, axis=0) - jnp.take(h_e, rev, axis=0)
        h_e = _depth_step(h_e, m_excl, wmsg_t, wih_t, whh_t, bih, bhh, lnw, lnb)

    m_to_atom = jax.ops.segment_sum(h_e, dst_dir, num_segments=N)
    h_atom = jax.nn.relu(jnp.concatenate([a_all, m_to_atom], axis=1) @ atom_out_W.T + atom_out_b)
    h_atom = _ln(h_atom, ln_a_w, ln_a_b)
    h_graph = jax.ops.segment_sum(h_atom, batch, num_segments=G)
    out = h_graph @ readout_W.T + readout_b
    return (out.astype(jnp.float64), h_atom.astype(jnp.float64), h_e.astype(jnp.float64))


# speedup vs baseline: 47.2690x; 1.4994x over previous
import functools

import jax
import jax.numpy as jnp
from jax import lax
from jax.experimental import pallas as pl
from jax.experimental.pallas import tpu as pltpu
from jax.experimental.pallas import tpu_sc as plsc

N = 10000
E = 320000
E2 = 2 * E
H = 128
G = 256
DEPTH = 3
TASKS = 12

EDGE_BLOCK = 2560
NBLK = E2 // EDGE_BLOCK
SHIFT = E // EDGE_BLOCK

NC, NS, L = 2, 16, 16
NW = NC * NS
GRP = 128
CHUNK = 256
NGRP = CHUNK // GRP
NCHUNK_TOT = E2 // CHUNK
ITERS = -(-NCHUNK_TOT // NW)
TROW_A = 624
TROW_LAST = N - 15 * TROW_A

_sc_mesh = plsc.VectorSubcoreMesh(core_axis_name="c", subcore_axis_name="s")


def _seg_partials_body(vals_hbm, idx_hbm, out_hbm, hecopy_hbm, table_sh, vals_v, idx_v):
    i32 = jnp.int32
    c = lax.axis_index("c")
    s = lax.axis_index("s")
    wid = c * i32(NS) + s

    @pl.loop(jnp.int32(0), jnp.int32(CHUNK))
    def _zrow(r):
        for k in range(H // L):
            vals_v[r, pl.ds(k * L, L)] = jnp.zeros((L,), jnp.float32)

    trow = pl.multiple_of(s * i32(TROW_A), 8)

    @pl.when(s < i32(NS - 1))
    def _z0():
        for o in (0, 256, 512):
            pltpu.sync_copy(vals_v.at[pl.ds(0, min(256, TROW_A - o))],
                            table_sh.at[pl.ds(trow + i32(o), min(256, TROW_A - o))])

    @pl.when(s == i32(NS - 1))
    def _z1():
        for o in (0, 256, 512):
            pltpu.sync_copy(vals_v.at[pl.ds(0, min(256, TROW_LAST - o))],
                            table_sh.at[pl.ds(i32(15 * TROW_A + o), min(256, TROW_LAST - o))])

    plsc.subcore_barrier()

    @pl.loop(jnp.int32(0), jnp.int32(ITERS))
    def _chunk(t):
        cid = t * i32(NW) + wid

        @pl.when(cid < i32(NCHUNK_TOT))
        def _do():
            off = pl.multiple_of(cid * i32(CHUNK), CHUNK)
            pltpu.sync_copy(vals_hbm.at[pl.ds(off, CHUNK)], vals_v)
            pltpu.sync_copy(idx_hbm.at[pl.ds(pl.multiple_of(cid * i32(NGRP), NGRP), NGRP)],
                            idx_v)
            pltpu.sync_copy(vals_v, hecopy_hbm.at[pl.ds(off, CHUNK)])
            for j in range(NGRP):
                pltpu.sync_copy(vals_v.at[pl.ds(j * GRP, GRP)],
                                table_sh.at[idx_v.at[jnp.int32(j)]], add=True)

    plsc.subcore_barrier()

    @pl.when(s < i32(NS - 1))
    def _w0():
        pltpu.sync_copy(table_sh.at[pl.ds(trow, TROW_A)],
                        out_hbm.at[pl.ds(pl.multiple_of(c * i32(N) + trow, 8), TROW_A)])

    @pl.when(s == i32(NS - 1))
    def _w1():
        pltpu.sync_copy(table_sh.at[pl.ds(i32(15 * TROW_A), TROW_LAST)],
                        out_hbm.at[pl.ds(pl.multiple_of(c * i32(N) + i32(15 * TROW_A), 8), TROW_LAST)])


_seg_partials = pl.kernel(
    _seg_partials_body,
    name="seg_partials",
    out_type=(jax.ShapeDtypeStruct((2 * N, H), jnp.float32),
              jax.ShapeDtypeStruct((E2, H), jnp.float32)),
    mesh=_sc_mesh,
    compiler_params=pltpu.CompilerParams(use_tc_tiling_on_sc=False),
    scratch_types=[
        pltpu.VMEM_SHARED((N, H), jnp.float32),
        pltpu.VMEM((CHUNK, H), jnp.float32),
        pltpu.VMEM((NGRP, GRP), jnp.int32),
    ],
)


def _gather_body(patch, *refs):
    if patch:
        (table_hbm, src_hbm, rev_hbm, he_hbm, q_hbm,
         q_v, src_v, gr_v) = refs
    else:
        table_hbm, src_hbm, q_hbm, q_v, src_v = refs
    i32 = jnp.int32
    c = lax.axis_index("c")
    s = lax.axis_index("s")
    wid = c * i32(NS) + s

    @pl.loop(jnp.int32(0), jnp.int32(ITERS))
    def _chunk(t):
        cid = t * i32(NW) + wid

        @pl.when(cid < i32(NCHUNK_TOT))
        def _do():
            off = pl.multiple_of(cid * i32(CHUNK), CHUNK)
            pltpu.sync_copy(
                src_hbm.at[pl.ds(pl.multiple_of(cid * i32(NGRP), NGRP), NGRP)],
                src_v)
            for j in range(NGRP):
                pltpu.sync_copy(table_hbm.at[src_v.at[jnp.int32(j)]],
                                q_v.at[pl.ds(j * GRP, GRP)])
            if patch:
                pltpu.sync_copy(
                    rev_hbm.at[pl.ds(pl.multiple_of(cid * i32(NGRP), NGRP), NGRP)],
                    src_v)
                for j in range(NGRP):
                    pltpu.sync_copy(he_hbm.at[src_v.at[jnp.int32(j)]],
                                    gr_v.at[pl.ds(j * GRP, GRP)])

                @pl.loop(jnp.int32(0), jnp.int32(CHUNK))
                def _row(r):
                    for k in range(H // L):
                        sl = pl.ds(k * L, L)
                        q_v[r, sl] = q_v[r, sl] - gr_v[r, sl]

            pltpu.sync_copy(q_v, q_hbm.at[pl.ds(off, CHUNK)])


_gather_patch = pl.kernel(
    functools.partial(_gather_body, True),
    name="gather_patch",
    out_type=jax.ShapeDtypeStruct((E2, H), jnp.float32),
    mesh=_sc_mesh,
    compiler_params=pltpu.CompilerParams(use_tc_tiling_on_sc=False),
    scratch_types=[
        pltpu.VMEM((CHUNK, H), jnp.float32),
        pltpu.VMEM((NGRP, GRP), jnp.int32),
        pltpu.VMEM((CHUNK, H), jnp.float32),
    ],
)

_gather_plain = pl.kernel(
    functools.partial(_gather_body, False),
    name="gather_plain",
    out_type=jax.ShapeDtypeStruct((E2, H), jnp.float32),
    mesh=_sc_mesh,
    compiler_params=pltpu.CompilerParams(use_tc_tiling_on_sc=False),
    scratch_types=[
        pltpu.VMEM((CHUNK, H), jnp.float32),
        pltpu.VMEM((NGRP, GRP), jnp.int32),
    ],
)


AROW = 250
ANCH = N // AROW
AITER = -(-ANCH // NW)


def _add_parts_body(parts_hbm, out_hbm, a_v, b_v):
    i32 = jnp.int32
    c = lax.axis_index("c")
    s = lax.axis_index("s")
    wid = c * i32(NS) + s

    @pl.loop(jnp.int32(0), jnp.int32(AITER))
    def _chunk(t):
        cid = t * i32(NW) + wid

        @pl.when(cid < i32(ANCH))
        def _do():
            r0 = cid * i32(AROW)
            pltpu.sync_copy(parts_hbm.at[pl.ds(r0, AROW)], a_v)
            pltpu.sync_copy(parts_hbm.at[pl.ds(r0 + i32(N), AROW)], b_v)

            @pl.loop(jnp.int32(0), jnp.int32(AROW))
            def _row(r):
                for k in range(H // L):
                    sl = pl.ds(k * L, L)
                    a_v[r, sl] = a_v[r, sl] + b_v[r, sl]

            pltpu.sync_copy(a_v, out_hbm.at[pl.ds(r0, AROW)])


_add_parts = pl.kernel(
    _add_parts_body,
    name="add_parts",
    out_type=jax.ShapeDtypeStruct((N, H), jnp.float32),
    mesh=_sc_mesh,
    compiler_params=pltpu.CompilerParams(use_tc_tiling_on_sc=False),
    scratch_types=[
        pltpu.VMEM((AROW, H), jnp.float32),
        pltpu.VMEM((AROW, H), jnp.float32),
    ],
)


def _depth_step_body(h_ref, q_ref, wmsg_ref, wih_ref, whh_ref,
                     bih_ref, bhh_ref, lnw_ref, lnb_ref, out_ref):
    h_e = h_ref[...]
    m = jnp.dot(q_ref[...], wmsg_ref[...], preferred_element_type=jnp.float32)
    h_new = jnp.maximum(h_e + m, 0.0)
    gi = jnp.dot(h_new, wih_ref[...], preferred_element_type=jnp.float32) + bih_ref[...]
    gh = jnp.dot(h_e, whh_ref[...], preferred_element_type=jnp.float32) + bhh_ref[...]
    r = jax.nn.sigmoid(gi[:, :H] + gh[:, :H])
    z = jax.nn.sigmoid(gi[:, H:2 * H] + gh[:, H:2 * H])
    ng = jnp.tanh(gi[:, 2 * H:] + r * gh[:, 2 * H:])
    h = (1.0 - z) * ng + z * h_e
    mu = jnp.mean(h, axis=-1, keepdims=True)
    var = jnp.mean((h - mu) ** 2, axis=-1, keepdims=True)
    out_ref[...] = (h - mu) * jax.lax.rsqrt(var + 1e-5) * lnw_ref[...] + lnb_ref[...]


def _depth_step(h_e, q, wmsg_t, wih_t, whh_t, bih, bhh, lnw, lnb):
    blk = lambda i: (i, i * 0)
    full = lambda i: (i * 0, i * 0)
    call = pl.pallas_call(
        _depth_step_body,
        name="depth_step",
        grid=(NBLK,),
        in_specs=[
            pl.BlockSpec((EDGE_BLOCK, H), blk),
            pl.BlockSpec((EDGE_BLOCK, H), blk),
            pl.BlockSpec((H, H), full),
            pl.BlockSpec((H, 3 * H), full),
            pl.BlockSpec((H, 3 * H), full),
            pl.BlockSpec((1, 3 * H), full),
            pl.BlockSpec((1, 3 * H), full),
            pl.BlockSpec((1, H), full),
            pl.BlockSpec((1, H), full),
        ],
        out_specs=pl.BlockSpec((EDGE_BLOCK, H), blk),
        out_shape=jax.ShapeDtypeStruct((E2, H), jnp.float32),
    )
    return call(h_e, q, wmsg_t, wih_t, whh_t, bih, bhh, lnw, lnb)


def _init_edges_body(hsrc_ref, eall_ref, lnw_ref, lnb_ref, out_ref):
    h = jnp.maximum(hsrc_ref[...] + eall_ref[...], 0.0)
    mu = jnp.mean(h, axis=-1, keepdims=True)
    var = jnp.mean((h - mu) ** 2, axis=-1, keepdims=True)
    out_ref[...] = (h - mu) * jax.lax.rsqrt(var + 1e-5) * lnw_ref[...] + lnb_ref[...]


def _init_edges(h_src, e_all, lnw, lnb):
    blk = lambda i: (i, i * 0)
    ewrap = lambda i: (i % SHIFT, i * 0)
    full = lambda i: (i * 0, i * 0)
    call = pl.pallas_call(
        _init_edges_body,
        name="init_edges",
        grid=(NBLK,),
        in_specs=[
            pl.BlockSpec((EDGE_BLOCK, H), blk),
            pl.BlockSpec((EDGE_BLOCK, H), ewrap),
            pl.BlockSpec((1, H), full),
            pl.BlockSpec((1, H), full),
        ],
        out_specs=pl.BlockSpec((EDGE_BLOCK, H), blk),
        out_shape=jax.ShapeDtypeStruct((E2, H), jnp.float32),
    )
    return call(h_src, e_all, lnw, lnb)


def _build_directed(edge_index):
    src_u, dst_u = edge_index[0], edge_index[1]
    src_dir = jnp.concatenate([src_u, dst_u], axis=0)
    dst_dir = jnp.concatenate([dst_u, src_u], axis=0)
    key = (src_dir.astype(jnp.int64) << 32) + dst_dir.astype(jnp.int64)
    rev_key = (dst_dir.astype(jnp.int64) << 32) + src_dir.astype(jnp.int64)
    order = jnp.argsort(key)
    key_sorted = key[order]
    idx_in_sorted = jnp.searchsorted(key_sorted, rev_key)
    rev = order[idx_in_sorted]
    return src_dir.astype(jnp.int32), dst_dir.astype(jnp.int32), rev.astype(jnp.int32)


def _ln(x, w, b):
    mu = jnp.mean(x, axis=-1, keepdims=True)
    var = jnp.mean((x - mu) ** 2, axis=-1, keepdims=True)
    return (x - mu) / jnp.sqrt(var + 1e-5) * w + b


def kernel(x_z, x_atom, edge_index, edge_attr, batch, emb_z, atom_in_W, atom_in_b,
           bond_in_W, bond_in_b, W_msg, gru_W_ih, gru_W_hh, gru_b_ih, gru_b_hh,
           atom_out_W, atom_out_b, readout_W, readout_b, ln_e_w, ln_e_b, ln_a_w, ln_a_b):
    src_dir, dst_dir, rev = _build_directed(edge_index)
    src2 = src_dir.reshape(E2 // GRP, GRP)
    dst2 = dst_dir.reshape(E2 // GRP, GRP)
    rev2 = rev.reshape(E2 // GRP, GRP)

    atom_in_W = atom_in_W.astype(jnp.float32)
    bond_in_W = bond_in_W.astype(jnp.float32)
    W_msg = W_msg.astype(jnp.float32)
    gru_W_ih = gru_W_ih.astype(jnp.float32)
    gru_W_hh = gru_W_hh.astype(jnp.float32)
    atom_out_W = atom_out_W.astype(jnp.float32)
    readout_W = readout_W.astype(jnp.float32)

    a_embed = jnp.take(emb_z, x_z, axis=0)
    a_all = jnp.concatenate([a_embed, x_atom], axis=1)
    h_all = a_all @ atom_in_W.T + atom_in_b
    e_all = edge_attr @ bond_in_W.T + bond_in_b

    h_src = _gather_plain(h_all, src2)
    h_e = _init_edges(h_src, e_all, ln_e_w.reshape(1, H), ln_e_b.reshape(1, H))

    wmsg_t = W_msg.T
    wih_t = gru_W_ih.T
    whh_t = gru_W_hh.T
    bih = gru_b_ih.reshape(1, 3 * H)
    bhh = gru_b_hh.reshape(1, 3 * H)
    lnw = ln_e_w.reshape(1, H)
    lnb = ln_e_b.reshape(1, H)

    for _ in range(DEPTH):
        parts, he_sc = _seg_partials(h_e, dst2)
        m_in = _add_parts(parts)
        q = _gather_patch(m_in, src2, rev2, he_sc)
        h_e = _depth_step(h_e, q, wmsg_t, wih_t, whh_t, bih, bhh, lnw, lnb)

    parts, _ = _seg_partials(h_e, dst2)
    m_to_atom = _add_parts(parts)
    h_atom = jax.nn.relu(jnp.concatenate([a_all, m_to_atom], axis=1) @ atom_out_W.T + atom_out_b)
    h_atom = _ln(h_atom, ln_a_w, ln_a_b)
    h_graph = jax.ops.segment_sum(h_atom, batch, num_segments=G)
    out = h_graph @ readout_W.T + readout_b
    return (out.astype(jnp.float64), h_atom.astype(jnp.float64), h_e.astype(jnp.float64))
